# baseline (speedup 1.0000x reference)
import functools

import jax
import jax.numpy as jnp
from jax import lax
from jax.experimental import pallas as pl
from jax.experimental.pallas import tpu as pltpu
from jax.experimental.pallas import tpu_sc as plsc

L = 16
NC = 2
NS = 16
NW = NC * NS
BATCH = 16384
BPW = BATCH // NW
CHUNK = 128
NCHUNK = BPW // CHUNK
NGROUP = BPW // L
EMB = 64

_MESH = plsc.VectorSubcoreMesh(
    core_axis_name="c", subcore_axis_name="s", num_cores=NC, num_subcores=NS
)


def _inv_norm(n2):
    i = plsc.bitcast(n2, jnp.int32)
    y = plsc.bitcast(0x5F3759DF - (i >> 1), jnp.float32)
    for _ in range(3):
        y = y * (1.5 - 0.5 * n2 * y * y)
    norm = n2 * y
    return jnp.where(norm > 1e-12, y, jnp.float32(1e12))


@functools.partial(
    pl.kernel,
    out_type=jax.ShapeDtypeStruct((BATCH,), jnp.float32),
    mesh=_MESH,
    compiler_params=pltpu.CompilerParams(
        needs_layout_passes=False, use_tc_tiling_on_sc=False
    ),
    scratch_types=[
        pltpu.VMEM((NCHUNK, CHUNK), jnp.int32),
        pltpu.VMEM((NCHUNK, CHUNK), jnp.int32),
        pltpu.VMEM((NCHUNK, CHUNK), jnp.int32),
        pltpu.VMEM((BPW, EMB), jnp.float32),
        pltpu.VMEM((BPW, EMB), jnp.float32),
        pltpu.VMEM((BPW, EMB), jnp.float32),
        pltpu.VMEM((BPW,), jnp.float32),
        pltpu.SemaphoreType.DMA,
    ],
)
def _sc_kernel(hidx_hbm, ridx_hbm, tidx_hbm, ent_hbm, rel_hbm, out_hbm,
               hi_v, ri_v, ti_v, hrow, rrow, trow, res_v, sem):
    wid = lax.axis_index("s") * NC + lax.axis_index("c")
    pltpu.sync_copy(hidx_hbm.at[wid], hi_v)
    pltpu.sync_copy(ridx_hbm.at[wid], ri_v)
    pltpu.sync_copy(tidx_hbm.at[wid], ti_v)

    copies = []
    for j in range(NCHUNK):
        sl = pl.ds(j * CHUNK, CHUNK)
        copies.append(pltpu.async_copy(ent_hbm.at[hi_v.at[j]], hrow.at[sl], sem))
        copies.append(pltpu.async_copy(rel_hbm.at[ri_v.at[j]], rrow.at[sl], sem))
        copies.append(pltpu.async_copy(ent_hbm.at[ti_v.at[j]], trow.at[sl], sem))
    for c in copies:
        c.wait()

    zeros = jnp.zeros((L,), jnp.float32)

    def group(g, carry):
        rid = lax.iota(jnp.int32, L) + g * L
        acc_h = zeros
        acc_r = zeros
        acc_t = zeros
        for d in range(EMB):
            dv = jnp.full((L,), d, jnp.int32)
            hv = plsc.load_gather(hrow, [rid, dv])
            rv = plsc.load_gather(rrow, [rid, dv])
            tv = plsc.load_gather(trow, [rid, dv])
            acc_h = acc_h + hv * hv
            acc_r = acc_r + rv * rv
            acc_t = acc_t + tv * tv
        ih = _inv_norm(acc_h)
        ir = _inv_norm(acc_r)
        it = _inv_norm(acc_t)
        acc = zeros
        for d in range(EMB):
            dv = jnp.full((L,), d, jnp.int32)
            hv = plsc.load_gather(hrow, [rid, dv])
            rv = plsc.load_gather(rrow, [rid, dv])
            tv = plsc.load_gather(trow, [rid, dv])
            acc = acc + jnp.abs(hv * ih + rv * ir - tv * it)
        res_v[pl.ds(g * L, L)] = acc
        return carry

    lax.fori_loop(0, NGROUP, group, 0)
    pltpu.sync_copy(res_v, out_hbm.at[pl.ds(wid * BPW, BPW)])


def kernel(edge, entity_embedding, relation_embedding):
    edge = edge.astype(jnp.int32)
    hidx = edge[:, 0].reshape(NW, NCHUNK, CHUNK)
    ridx = edge[:, 1].reshape(NW, NCHUNK, CHUNK)
    tidx = edge[:, 2].reshape(NW, NCHUNK, CHUNK)
    return _sc_kernel(hidx, ridx, tidx, entity_embedding, relation_embedding)
